# baseline (speedup 1.0000x reference)
import functools

import jax
import jax.numpy as jnp
from jax import lax
from jax.experimental import pallas as pl
from jax.experimental.pallas import tpu as pltpu
from jax.experimental.pallas import tpu_sc as plsc

B = 16384
V = 1000000
D = 32
NNEG = 20
NC = 2
NS = 16
DPC = D // NC
RPT = B // NS
STG = 8
VCH = V // STG
NGRP = 5

_mesh = plsc.VectorSubcoreMesh(core_axis_name="c", subcore_axis_name="s")


@functools.partial(
    pl.kernel,
    mesh=_mesh,
    compiler_params=pltpu.CompilerParams(
        needs_layout_passes=False, use_tc_tiling_on_sc=False
    ),
    out_type=(
        jax.ShapeDtypeStruct((NC * B,), jnp.float32),
        jax.ShapeDtypeStruct((NC * NNEG * B,), jnp.float32),
    ),
    scratch_types=[
        pltpu.VMEM_SHARED((V,), jnp.float32),
        pltpu.VMEM((RPT,), jnp.int32),
        pltpu.VMEM((RPT,), jnp.int32),
        pltpu.VMEM((NNEG * RPT,), jnp.int32),
        pltpu.VMEM((DPC * RPT,), jnp.float32),
        pltpu.VMEM((RPT,), jnp.float32),
        pltpu.VMEM((NGRP * RPT,), jnp.float32),
        pltpu.VMEM((RPT,), jnp.float32),
        pltpu.VMEM((NNEG * RPT,), jnp.float32),
        pltpu.SemaphoreType.DMA,
        pltpu.SemaphoreType.DMA,
    ],
)
def _sc_dots(center_hbm, context_hbm, negtf_hbm, int_hbm, outt_hbm,
             posd_hbm, negd_hbm,
             rowbuf, cidx, tidx, nidx, cvals, tvals, nvals, pacc, nacc,
             sem0, sem1):
    c = lax.axis_index("c")
    s = lax.axis_index("s")
    rbase = s * RPT

    pltpu.sync_copy(center_hbm.at[pl.ds(rbase, RPT)], cidx)
    pltpu.sync_copy(context_hbm.at[pl.ds(rbase, RPT)], tidx)
    for n in range(NNEG):
        pltpu.sync_copy(
            negtf_hbm.at[pl.ds(n * B + rbase, RPT)],
            nidx.at[pl.ds(n * RPT, RPT)],
        )

    zero16 = jnp.zeros((16,), jnp.float32)

    def zero_body(rv, _):
        pacc[pl.ds(rv * 16, 16)] = zero16
        for n in range(NNEG):
            nacc[pl.ds(n * RPT + rv * 16, 16)] = zero16
        return 0

    lax.fori_loop(0, RPT // 16, zero_body, 0)

    def stage_row(table_hbm, gd):
        @pl.when(s < STG)
        def _():
            off = s * VCH
            pltpu.async_copy(
                table_hbm.at[gd, pl.ds(off, VCH)],
                rowbuf.at[pl.ds(off, VCH)],
                sem0,
            ).wait()
        plsc.subcore_barrier()

    def in_body(d, _):
        stage_row(int_hbm, c * DPC + d)
        pltpu.sync_copy(rowbuf.at[cidx], cvals.at[pl.ds(d * RPT, RPT)])
        plsc.subcore_barrier()
        return 0

    lax.fori_loop(0, DPC, in_body, 0)

    def out_body(d, _):
        stage_row(outt_hbm, c * DPC + d)
        pltpu.sync_copy(rowbuf.at[tidx], tvals)

        def pos_body(rv, _):
            r16 = rv * 16
            cv = cvals[pl.ds(d * RPT + r16, 16)]
            pacc[pl.ds(r16, 16)] = pacc[pl.ds(r16, 16)] + cv * tvals[pl.ds(r16, 16)]
            return 0

        lax.fori_loop(0, RPT // 16, pos_body, 0)

        for g in range(NNEG // NGRP):
            cps = [
                pltpu.async_copy(
                    rowbuf.at[nidx.at[pl.ds((g * NGRP + n) * RPT, RPT)]],
                    nvals.at[pl.ds(n * RPT, RPT)],
                    sem1,
                )
                for n in range(NGRP)
            ]
            for cp in cps:
                cp.wait()

            def fma_body(rv, _, g=g):
                r16 = rv * 16
                cv = cvals[pl.ds(d * RPT + r16, 16)]
                for n in range(NGRP):
                    o = (g * NGRP + n) * RPT + r16
                    i = n * RPT + r16
                    nacc[pl.ds(o, 16)] = nacc[pl.ds(o, 16)] + cv * nvals[pl.ds(i, 16)]
                return 0

            lax.fori_loop(0, RPT // 16, fma_body, 0)
        plsc.subcore_barrier()
        return 0

    lax.fori_loop(0, DPC, out_body, 0)

    pltpu.sync_copy(pacc, posd_hbm.at[pl.ds(c * B + rbase, RPT)])
    for n in range(NNEG):
        pltpu.sync_copy(
            nacc.at[pl.ds(n * RPT, RPT)],
            negd_hbm.at[pl.ds(c * (NNEG * B) + n * B + rbase, RPT)],
        )


def _loss_body(posd_ref, negd_ref, out_ref):
    pos_dot = posd_ref[0] + posd_ref[1]
    neg_dot = negd_ref[0:NNEG, :] + negd_ref[NNEG:2 * NNEG, :]
    pos = 1.0 / (1.0 + jnp.exp(-pos_dot))
    negs = jnp.sum(1.0 / (1.0 + jnp.exp(neg_dot)), axis=0)
    total = jnp.sum(jnp.log(pos)) + jnp.sum(jnp.log(negs))
    out_ref[0, 0] = -total / B


_finish = pl.pallas_call(
    _loss_body,
    out_shape=jax.ShapeDtypeStruct((1, 1), jnp.float32),
    out_specs=pl.BlockSpec(memory_space=pltpu.SMEM),
)


def kernel(center, context, negative, in_embed, out_embed):
    negtf = negative.T.reshape(-1)
    posd, negd = _sc_dots(center, context, negtf, in_embed.T, out_embed.T)
    loss = _finish(posd.reshape(NC, B), negd.reshape(NC * NNEG, B))
    return loss[0, 0]


# speedup vs baseline: 5.0737x; 5.0737x over previous
import functools

import jax
import jax.numpy as jnp
from jax import lax
from jax.experimental import pallas as pl
from jax.experimental.pallas import tpu as pltpu
from jax.experimental.pallas import tpu_sc as plsc

B = 16384
D = 32
NNEG = 20
NC = 2
NS = 16
NW = NC * NS
RPW = B // NW
CH = 64
NCH = RPW // CH
CHN = CH * NNEG
NBLK = CH // 16

_mesh = plsc.VectorSubcoreMesh(core_axis_name="c", subcore_axis_name="s")


@functools.partial(
    pl.kernel,
    mesh=_mesh,
    compiler_params=pltpu.CompilerParams(
        needs_layout_passes=False, use_tc_tiling_on_sc=False
    ),
    out_type=(
        jax.ShapeDtypeStruct((B,), jnp.float32),
        jax.ShapeDtypeStruct((B,), jnp.float32),
    ),
    scratch_types=[
        pltpu.VMEM((RPW,), jnp.int32),
        pltpu.VMEM((RPW,), jnp.int32),
        pltpu.VMEM((RPW * NNEG,), jnp.int32),
        pltpu.VMEM((RPW, D), jnp.float32),
        pltpu.VMEM((RPW, D), jnp.float32),
        pltpu.VMEM((2 * CHN, D), jnp.float32),
        pltpu.VMEM((RPW,), jnp.float32),
        pltpu.VMEM((RPW,), jnp.float32),
        pltpu.SemaphoreType.DMA,
        pltpu.SemaphoreType.DMA,
        pltpu.SemaphoreType.DMA,
        pltpu.SemaphoreType.DMA,
    ],
)
def _sc_scores(center_hbm, context_hbm, negflat_hbm, in_hbm, out_hbm,
               pos_hbm, negsum_hbm,
               cidx, tidx, nidx, crow, trow, nrow, posb, negb,
               sem0, sem1, sem2, sem3):
    wid = lax.axis_index("s") * NC + lax.axis_index("c")
    base = wid * RPW

    pltpu.sync_copy(center_hbm.at[pl.ds(base, RPW)], cidx)
    pltpu.sync_copy(context_hbm.at[pl.ds(base, RPW)], tidx)
    pltpu.sync_copy(negflat_hbm.at[pl.ds(base * NNEG, RPW * NNEG)], nidx)

    cp0 = pltpu.async_copy(in_hbm.at[cidx], crow, sem0)
    cp1 = pltpu.async_copy(out_hbm.at[tidx], trow, sem1)
    pltpu.async_copy(
        out_hbm.at[nidx.at[pl.ds(0, CHN)]], nrow.at[pl.ds(0, CHN), :], sem2
    )
    cp0.wait()
    cp1.wait()

    lane = lax.iota(jnp.int32, 16)

    def chunk_body(ch, _):
        even = (ch & 1) == 0
        half = (ch & 1) * CHN

        @pl.when(jnp.logical_and(ch + 1 < NCH, even))
        def _():
            pltpu.async_copy(
                out_hbm.at[nidx.at[pl.ds((ch + 1) * CHN, CHN)]],
                nrow.at[pl.ds(CHN, CHN), :],
                sem3,
            )

        @pl.when(jnp.logical_and(ch + 1 < NCH, jnp.logical_not(even)))
        def _():
            pltpu.async_copy(
                out_hbm.at[nidx.at[pl.ds((ch + 1) * CHN, CHN)]],
                nrow.at[pl.ds(0, CHN), :],
                sem2,
            )

        @pl.when(even)
        def _():
            pltpu.make_async_copy(
                out_hbm.at[nidx.at[pl.ds(ch * CHN, CHN)]],
                nrow.at[pl.ds(0, CHN), :],
                sem2,
            ).wait()

        @pl.when(jnp.logical_not(even))
        def _():
            pltpu.make_async_copy(
                out_hbm.at[nidx.at[pl.ds(ch * CHN, CHN)]],
                nrow.at[pl.ds(CHN, CHN), :],
                sem3,
            ).wait()

        def blk_body(blk, _):
            crow_idx = blk * 16 + lane
            grow_idx = ch * CH + crow_idx
            pair0 = half + crow_idx * NNEG
            accp = jnp.zeros((16,), jnp.float32)
            accn = [jnp.zeros((16,), jnp.float32) for _ in range(NNEG)]
            for d in range(D):
                dsp = jnp.full((16,), d, jnp.int32)
                cg = plsc.load_gather(crow, [grow_idx, dsp])
                tg = plsc.load_gather(trow, [grow_idx, dsp])
                accp = accp + cg * tg
                for n in range(NNEG):
                    gn = plsc.load_gather(nrow, [pair0 + n, dsp])
                    accn[n] = accn[n] + gn * cg
            posv = 1.0 / (1.0 + jnp.exp(-accp))
            negv = jnp.zeros((16,), jnp.float32)
            for n in range(NNEG):
                negv = negv + 1.0 / (1.0 + jnp.exp(accn[n]))
            r0 = ch * CH + blk * 16
            posb[pl.ds(r0, 16)] = posv
            negb[pl.ds(r0, 16)] = negv
            return 0

        lax.fori_loop(0, NBLK, blk_body, 0)
        return 0

    lax.fori_loop(0, NCH, chunk_body, 0)

    pltpu.sync_copy(posb, pos_hbm.at[pl.ds(base, RPW)])
    pltpu.sync_copy(negb, negsum_hbm.at[pl.ds(base, RPW)])


def _loss_body(pos_ref, neg_ref, out_ref):
    total = jnp.sum(jnp.log(pos_ref[...])) + jnp.sum(jnp.log(neg_ref[...]))
    out_ref[0, 0] = -total / B


_finish = pl.pallas_call(
    _loss_body,
    out_shape=jax.ShapeDtypeStruct((1, 1), jnp.float32),
    out_specs=pl.BlockSpec(memory_space=pltpu.SMEM),
)


def kernel(center, context, negative, in_embed, out_embed):
    negflat = negative.reshape(-1)
    pos, neg = _sc_scores(center, context, negflat, in_embed, out_embed)
    loss = _finish(pos.reshape(128, 128), neg.reshape(128, 128))
    return loss[0, 0]
